# Initial kernel scaffold; baseline (speedup 1.0000x reference)
import jax
import jax.numpy as jnp
from jax.experimental import pallas as pl


def kernel(X, I, attn_kernel):
    raise NotImplementedError("write your pallas kernel here")


# speedup vs baseline: 4.8675x; 4.8675x over previous
import functools

import jax
import jax.numpy as jnp
from jax import lax
from jax.experimental import pallas as pl
from jax.experimental.pallas import tpu as pltpu

N_NODES = 100000
D_FEAT = 256
NUM_GRAPHS = 512

BM = 1000
W = 64
NB = N_NODES // BM

NEG_INF = float("-inf")


def _attn_pool_kernel(x_ref, i_ref, a_ref, out_ref, stat_ref, *, interpret=False):
    k = pl.program_id(0)

    @pl.when(k == 0)
    def _init():
        out_ref[...] = jnp.zeros_like(out_ref)
        stat_ref[0] = jnp.float32(NEG_INF)
        stat_ref[1] = jnp.float32(0.0)

    x = x_ref[...]
    a = a_ref[...]
    c = jnp.dot(x, a, preferred_element_type=jnp.float32)

    m_old = stat_ref[0]
    m_blk = jnp.max(c)
    m_new = jnp.maximum(m_old, m_blk)
    alpha = jnp.exp(m_old - m_new)

    p = jnp.exp(c - m_new)
    stat_ref[0] = m_new
    stat_ref[1] = stat_ref[1] * alpha + jnp.sum(p)

    out_ref[...] = out_ref[...] * alpha

    weighted = p * x
    i_row = i_ref[0]

    def window_add(lo):
        base = jnp.minimum((lo // 8) * 8, NUM_GRAPHS - W)
        iota = lax.broadcasted_iota(jnp.int32, (W, BM), 0) + base
        hit = (iota == i_row) & (i_row >= lo)
        oh = hit.astype(jnp.float32)
        contrib = jnp.dot(oh, weighted, preferred_element_type=jnp.float32)
        out_ref[pl.ds(base, W), :] += contrib
        rem = jnp.where(i_row >= base + W, i_row, NUM_GRAPHS)
        return jnp.min(rem)

    lo = window_add(jnp.min(i_row))
    lax.while_loop(lambda l: l < NUM_GRAPHS, window_add, lo)

    @pl.when(k == NB - 1)
    def _finalize():
        out_ref[...] = out_ref[...] / stat_ref[1]


def kernel(X, I, attn_kernel):
    I32 = I.astype(jnp.int32).reshape(NB, 1, BM)
    grid = (NB,)
    return pl.pallas_call(
        functools.partial(_attn_pool_kernel),
        grid=grid,
        in_specs=[
            pl.BlockSpec((BM, D_FEAT), lambda i: (i, 0)),
            pl.BlockSpec((1, 1, BM), lambda i: (i, 0, 0)),
            pl.BlockSpec((D_FEAT, 1), lambda i: (0, 0)),
        ],
        out_specs=pl.BlockSpec((NUM_GRAPHS, D_FEAT), lambda i: (0, 0)),
        out_shape=jax.ShapeDtypeStruct((NUM_GRAPHS, D_FEAT), jnp.float32),
        scratch_shapes=[pltpu.SMEM((2,), jnp.float32)],
        compiler_params=pltpu.CompilerParams(
            dimension_semantics=("arbitrary",),
        ),
    )(X, I32, attn_kernel)


# speedup vs baseline: 6.4298x; 1.3210x over previous
import functools

import jax
import jax.numpy as jnp
from jax import lax
from jax.experimental import pallas as pl
from jax.experimental.pallas import tpu as pltpu

N_NODES = 100000
D_FEAT = 256
NUM_GRAPHS = 512

BM = 2000
W = 64
NB = N_NODES // BM

NEG_INF = float("-inf")


def _attn_pool_kernel(x_ref, i_ref, a_ref, out_ref, stat_ref, *, interpret=False):
    k = pl.program_id(0)

    @pl.when(k == 0)
    def _init():
        out_ref[...] = jnp.zeros_like(out_ref)
        stat_ref[0] = jnp.float32(NEG_INF)
        stat_ref[1] = jnp.float32(0.0)

    x = x_ref[...]
    a = a_ref[...]
    c = jnp.dot(x, a, preferred_element_type=jnp.float32)

    m_old = stat_ref[0]
    m_blk = jnp.max(c)
    m_new = jnp.maximum(m_old, m_blk)
    alpha = jnp.exp(m_old - m_new)

    p = jnp.exp(c - m_new)
    stat_ref[0] = m_new
    stat_ref[1] = stat_ref[1] * alpha + jnp.sum(p)

    @pl.when(m_blk > m_old)
    def _rescale():
        out_ref[...] = out_ref[...] * alpha

    weighted = p * x
    i_row = i_ref[0]

    def window_add(lo):
        base = jnp.minimum((lo // 8) * 8, NUM_GRAPHS - W)
        iota = lax.broadcasted_iota(jnp.int32, (W, BM), 0) + base
        hit = (iota == i_row) & (i_row >= lo)
        oh = hit.astype(jnp.float32)
        contrib = jnp.dot(oh, weighted, preferred_element_type=jnp.float32)
        out_ref[pl.ds(base, W), :] += contrib
        rem = jnp.where(i_row >= base + W, i_row, NUM_GRAPHS)
        return jnp.min(rem)

    lo = window_add(jnp.min(i_row))
    lax.while_loop(lambda l: l < NUM_GRAPHS, window_add, lo)

    @pl.when(k == NB - 1)
    def _finalize():
        out_ref[...] = out_ref[...] / stat_ref[1]


def kernel(X, I, attn_kernel):
    I32 = I.astype(jnp.int32).reshape(NB, 1, BM)
    grid = (NB,)
    return pl.pallas_call(
        functools.partial(_attn_pool_kernel),
        grid=grid,
        in_specs=[
            pl.BlockSpec((BM, D_FEAT), lambda i: (i, 0)),
            pl.BlockSpec((1, 1, BM), lambda i: (i, 0, 0)),
            pl.BlockSpec((D_FEAT, 1), lambda i: (0, 0)),
        ],
        out_specs=pl.BlockSpec((NUM_GRAPHS, D_FEAT), lambda i: (0, 0)),
        out_shape=jax.ShapeDtypeStruct((NUM_GRAPHS, D_FEAT), jnp.float32),
        scratch_shapes=[pltpu.SMEM((2,), jnp.float32)],
        compiler_params=pltpu.CompilerParams(
            dimension_semantics=("arbitrary",),
        ),
    )(X, I32, attn_kernel)


# speedup vs baseline: 7.4328x; 1.1560x over previous
import functools

import jax
import jax.numpy as jnp
from jax import lax
from jax.experimental import pallas as pl
from jax.experimental.pallas import tpu as pltpu

N_NODES = 100000
D_FEAT = 256
NUM_GRAPHS = 512

BM = 2000
W = 64
NB = N_NODES // BM

NEG_INF = float("-inf")


def _attn_pool_kernel(x_ref, i_ref, a_ref, out_ref, stat_ref, *, interpret=False):
    k = pl.program_id(0)

    @pl.when(k == 0)
    def _init():
        out_ref[...] = jnp.zeros_like(out_ref)
        stat_ref[0] = jnp.float32(NEG_INF)
        stat_ref[1] = jnp.float32(0.0)

    x = x_ref[...]
    a = a_ref[...]
    c = lax.dot_general(a, x, (((0,), (1,)), ((), ())),
                        preferred_element_type=jnp.float32)

    m_old = stat_ref[0]
    m_blk = jnp.max(c)
    m_new = jnp.maximum(m_old, m_blk)
    alpha = jnp.exp(m_old - m_new)

    p = jnp.exp(c - m_new)
    stat_ref[0] = m_new
    stat_ref[1] = stat_ref[1] * alpha + jnp.sum(p)

    @pl.when(m_blk > m_old)
    def _rescale():
        out_ref[...] = out_ref[...] * alpha

    i_row = i_ref[0]

    def window_add(lo):
        base = jnp.minimum((lo // 8) * 8, NUM_GRAPHS - W)
        iota = lax.broadcasted_iota(jnp.int32, (W, BM), 0) + base
        hit = (iota == i_row) & (i_row >= lo)
        ohp = jnp.where(hit, p, jnp.float32(0.0))
        contrib = jnp.dot(ohp, x, preferred_element_type=jnp.float32)
        out_ref[pl.ds(base, W), :] += contrib
        rem = jnp.where(i_row >= base + W, i_row, NUM_GRAPHS)
        return jnp.min(rem)

    lo = window_add(jnp.min(i_row))
    lax.while_loop(lambda l: l < NUM_GRAPHS, window_add, lo)

    @pl.when(k == NB - 1)
    def _finalize():
        out_ref[...] = out_ref[...] / stat_ref[1]


def kernel(X, I, attn_kernel):
    I32 = I.astype(jnp.int32).reshape(NB, 1, BM)
    grid = (NB,)
    return pl.pallas_call(
        functools.partial(_attn_pool_kernel),
        grid=grid,
        in_specs=[
            pl.BlockSpec((BM, D_FEAT), lambda i: (i, 0)),
            pl.BlockSpec((1, 1, BM), lambda i: (i, 0, 0)),
            pl.BlockSpec((D_FEAT, 1), lambda i: (0, 0)),
        ],
        out_specs=pl.BlockSpec((NUM_GRAPHS, D_FEAT), lambda i: (0, 0)),
        out_shape=jax.ShapeDtypeStruct((NUM_GRAPHS, D_FEAT), jnp.float32),
        scratch_shapes=[pltpu.SMEM((2,), jnp.float32)],
        compiler_params=pltpu.CompilerParams(
            dimension_semantics=("arbitrary",),
        ),
    )(X, I32, attn_kernel)


# speedup vs baseline: 7.6532x; 1.0297x over previous
import jax
import jax.numpy as jnp
from jax import lax
from jax.experimental import pallas as pl
from jax.experimental.pallas import tpu as pltpu

N_NODES = 100000
D_FEAT = 256
NUM_GRAPHS = 512

BM = 2000
W = 32
NB = N_NODES // BM

NEG_INF = float("-inf")


def _attn_pool_kernel(first_ref, last_ref, x_ref, i_ref, a_ref, out_ref,
                      stat_ref):
    k = pl.program_id(0)

    @pl.when(k == 0)
    def _init():
        out_ref[...] = jnp.zeros_like(out_ref)
        stat_ref[0] = jnp.float32(NEG_INF)
        stat_ref[1] = jnp.float32(0.0)

    x = x_ref[...]
    a = a_ref[...]
    c = lax.dot_general(a, x, (((0,), (1,)), ((), ())),
                        preferred_element_type=jnp.float32)

    m_old = stat_ref[0]
    m_blk = jnp.max(c)
    m_new = jnp.maximum(m_old, m_blk)
    alpha = jnp.exp(m_old - m_new)

    p = jnp.exp(c - m_new)
    stat_ref[0] = m_new
    stat_ref[1] = stat_ref[1] * alpha + jnp.sum(p)

    @pl.when(m_blk > m_old)
    def _rescale():
        out_ref[...] = out_ref[...] * alpha

    i_row = i_ref[0]
    first = first_ref[k]
    last = last_ref[k]

    base0 = pl.multiple_of(jnp.minimum((first // 8) * 8, NUM_GRAPHS - W), 8)
    iota = lax.broadcasted_iota(jnp.int32, (W, BM), 0)
    hit0 = (iota + base0) == i_row
    ohp0 = jnp.where(hit0, p, jnp.float32(0.0))
    contrib0 = jnp.dot(ohp0, x, preferred_element_type=jnp.float32)
    out_ref[pl.ds(base0, W), :] += contrib0

    def more(l):
        base = pl.multiple_of(jnp.minimum(l, NUM_GRAPHS - W), 8)
        hit = ((iota + base) == i_row) & (i_row >= l)
        ohp = jnp.where(hit, p, jnp.float32(0.0))
        contrib = jnp.dot(ohp, x, preferred_element_type=jnp.float32)
        out_ref[pl.ds(base, W), :] += contrib
        return base + W

    lax.while_loop(lambda l: l <= last, more, base0 + W)

    @pl.when(k == NB - 1)
    def _finalize():
        out_ref[...] = out_ref[...] / stat_ref[1]


def kernel(X, I, attn_kernel):
    I32 = I.astype(jnp.int32)
    first = I32[0::BM]
    last = I32[BM - 1::BM]
    I3 = I32.reshape(NB, 1, BM)
    grid_spec = pltpu.PrefetchScalarGridSpec(
        num_scalar_prefetch=2,
        grid=(NB,),
        in_specs=[
            pl.BlockSpec((BM, D_FEAT), lambda i, f, l: (i, 0)),
            pl.BlockSpec((1, 1, BM), lambda i, f, l: (i, 0, 0)),
            pl.BlockSpec((D_FEAT, 1), lambda i, f, l: (0, 0)),
        ],
        out_specs=pl.BlockSpec((NUM_GRAPHS, D_FEAT), lambda i, f, l: (0, 0)),
        scratch_shapes=[pltpu.SMEM((2,), jnp.float32)],
    )
    return pl.pallas_call(
        _attn_pool_kernel,
        grid_spec=grid_spec,
        out_shape=jax.ShapeDtypeStruct((NUM_GRAPHS, D_FEAT), jnp.float32),
        compiler_params=pltpu.CompilerParams(
            dimension_semantics=("arbitrary",),
        ),
    )(first, last, X, I3, attn_kernel)
